# baseline (speedup 1.0000x reference)
import jax
import jax.numpy as jnp
from jax.experimental import pallas as pl
from jax.experimental.pallas import tpu as pltpu

_INTERPRET = False


def _gat_layer_fm(xT, ea_l, mask, WpT, wsp, wdp, bp, heads, chp):
    hT = jnp.dot(WpT, xT, preferred_element_type=jnp.float32)
    s = jax.lax.dot_general(xT, wsp, (((0,), (0,)), ((), ())),
                            preferred_element_type=jnp.float32)
    dT = jax.lax.dot_general(wdp, xT, (((0,), (0,)), ((), ())),
                             preferred_element_type=jnp.float32)
    outs = []
    for h in range(heads):
        logit = s[:, h][:, None] + dT[h][None, :] + ea_l[h]
        logit = jnp.maximum(logit, 0.2 * logit)
        logit = jnp.where(mask, logit, jnp.float32(-1e9))
        m = jnp.max(logit, axis=0, keepdims=True)
        e = jnp.where(mask, jnp.exp(logit - m), 0.0)
        denom = jnp.maximum(jnp.sum(e, axis=0, keepdims=True),
                            jnp.float32(1e-30))
        oh = jnp.dot(hT[h * chp:(h + 1) * chp], e,
                     preferred_element_type=jnp.float32)
        outs.append(oh / denom)
    return jnp.concatenate(outs, axis=0) + bp


def _body(atomsT_ref, adjs_ref, et_ref,
          W1_ref, ws1_ref, wd1_ref, b1_ref,
          W2_ref, ws2_ref, wd2_ref, b2_ref,
          W3_ref, ws3_ref, wd3_ref, b3_ref,
          M_ref, out_ref, *, heads, chp, out_dim):
    xT = atomsT_ref[0]
    n = xT.shape[1]
    mask = adjs_ref[0] > 0.5
    ea_all = jax.lax.dot_general(M_ref[...], et_ref[...],
                                 (((1,), (0,)), ((), ())),
                                 preferred_element_type=jnp.float32)
    ea_all = ea_all.reshape(ea_all.shape[0], n, n)
    xT = _gat_layer_fm(xT, ea_all[0:heads], mask, W1_ref[...],
                       ws1_ref[...], wd1_ref[...], b1_ref[...], heads, chp)
    xT = _gat_layer_fm(xT, ea_all[heads:2 * heads], mask, W2_ref[...],
                       ws2_ref[...], wd2_ref[...], b2_ref[...], heads, chp)
    xT = _gat_layer_fm(xT, ea_all[2 * heads:2 * heads + 1], mask, W3_ref[...],
                       ws3_ref[...], wd3_ref[...], b3_ref[...], 1, chp)
    out_ref[0] = xT[:out_dim].T


def _fold(W, a):
    heads, ch = a.shape
    return jnp.einsum('dhc,hc->dh', W.reshape(W.shape[0], heads, ch), a)


def kernel(atoms, adjs, edges, W1, We1, as1, ad1, ae1, b1,
           W2, We2, as2, ad2, ae2, b2, W3, We3, as3, ad3, ae3, b3):
    Bb, n, dim = atoms.shape
    E = edges.shape[-1]
    heads, ch = as1.shape
    out_dim = W3.shape[1]
    chp = ((ch + 7) // 8) * 8
    outp = ((out_dim + 7) // 8) * 8
    pad_c = chp - ch

    W1pT = jnp.pad(W1.reshape(dim, heads, ch),
                   ((0, 0), (0, 0), (0, pad_c))).reshape(dim, heads * chp).T
    ws1p, wd1p = _fold(W1, as1), _fold(W1, ad1)
    b1p = jnp.pad(b1.reshape(heads, ch),
                  ((0, 0), (0, pad_c))).reshape(heads * chp)[:, None]
    W2p = jnp.pad(W2.reshape(heads, ch, heads, ch),
                  ((0, 0), (0, pad_c), (0, 0), (0, pad_c)))
    W2pT = W2p.reshape(heads * chp, heads * chp).T
    ws2p = jnp.pad(_fold(W2, as2).reshape(heads, ch, heads),
                   ((0, 0), (0, pad_c), (0, 0))).reshape(heads * chp, heads)
    wd2p = jnp.pad(_fold(W2, ad2).reshape(heads, ch, heads),
                   ((0, 0), (0, pad_c), (0, 0))).reshape(heads * chp, heads)
    b2p = jnp.pad(b2.reshape(heads, ch),
                  ((0, 0), (0, pad_c))).reshape(heads * chp)[:, None]
    W3p = jnp.pad(W3.reshape(heads, ch, out_dim),
                  ((0, 0), (0, pad_c), (0, 0))).reshape(heads * chp, out_dim)
    W3pT = jnp.pad(W3p, ((0, 0), (0, outp - out_dim))).T
    ws3p = jnp.pad(_fold(W3, as3).reshape(heads, ch, 1),
                   ((0, 0), (0, pad_c), (0, 0))).reshape(heads * chp, 1)
    wd3p = jnp.pad(_fold(W3, ad3).reshape(heads, ch, 1),
                   ((0, 0), (0, pad_c), (0, 0))).reshape(heads * chp, 1)
    b3p = jnp.pad(b3, (0, outp - out_dim))[:, None]
    V = jnp.concatenate([_fold(We1, ae1), _fold(We2, ae2), _fold(We3, ae3)],
                        axis=1)
    M = V.T.astype(jnp.bfloat16)

    atomsT = jnp.swapaxes(atoms, 1, 2)

    def full(x):
        return pl.BlockSpec(x.shape, lambda b: (0,) * x.ndim)

    weights = (W1pT, ws1p, wd1p, b1p, W2pT, ws2p, wd2p, b2p,
               W3pT, ws3p, wd3p, b3p, M)
    import functools
    body = functools.partial(_body, heads=heads, chp=chp, out_dim=out_dim)

    et = jnp.transpose(edges.astype(jnp.bfloat16).reshape(Bb * n * n, E),
                       (1, 0))
    out = pl.pallas_call(
        body,
        grid=(Bb,),
        in_specs=[
            pl.BlockSpec((1, dim, n), lambda b: (b, 0, 0)),
            pl.BlockSpec((1, n, n), lambda b: (b, 0, 0)),
            pl.BlockSpec((E, n * n), lambda b: (0, b)),
        ] + [full(w) for w in weights],
        out_specs=pl.BlockSpec((1, n, out_dim), lambda b: (b, 0, 0)),
        out_shape=jax.ShapeDtypeStruct((Bb, n, out_dim), jnp.float32),
        compiler_params=pltpu.CompilerParams(
            dimension_semantics=("parallel",)),
        interpret=_INTERPRET,
    )(atomsT, adjs, et, *weights)
    return out


# speedup vs baseline: 1.2335x; 1.2335x over previous
import jax
import jax.numpy as jnp
from jax.experimental import pallas as pl
from jax.experimental.pallas import tpu as pltpu

_INTERPRET = False


def _gat_layer_fm(xT, ea_l, mask, WpT, wsp, wdp, bp, heads, chp):
    hT = jnp.dot(WpT, xT, preferred_element_type=jnp.float32)
    s = jax.lax.dot_general(xT, wsp, (((0,), (0,)), ((), ())),
                            preferred_element_type=jnp.float32)
    dT = jax.lax.dot_general(wdp, xT, (((0,), (0,)), ((), ())),
                             preferred_element_type=jnp.float32)
    outs = []
    for h in range(heads):
        logit = s[:, h][:, None] + dT[h][None, :] + ea_l[h]
        logit = jnp.maximum(logit, 0.2 * logit)
        logit = jnp.where(mask, logit, jnp.float32(-1e9))
        m = jnp.max(logit, axis=0, keepdims=True)
        e = jnp.where(mask, jnp.exp(logit - m), 0.0)
        denom = jnp.maximum(jnp.sum(e, axis=0, keepdims=True),
                            jnp.float32(1e-30))
        oh = jnp.dot(hT[h * chp:(h + 1) * chp], e,
                     preferred_element_type=jnp.float32)
        outs.append(oh / denom)
    return jnp.concatenate(outs, axis=0) + bp


def _body(atomsT_ref, adjs_ref, et_ref,
          W1_ref, ws1_ref, wd1_ref, b1_ref,
          W2_ref, ws2_ref, wd2_ref, b2_ref,
          W3_ref, ws3_ref, wd3_ref, b3_ref,
          M_ref, out_ref, *, heads, chp, out_dim):
    for g in range(out_ref.shape[0]):
        xT = atomsT_ref[g]
        n = xT.shape[1]
        mask = adjs_ref[g] > 0.5
        ea_all = jax.lax.dot_general(M_ref[...], et_ref[g],
                                     (((1,), (0,)), ((), ())),
                                     preferred_element_type=jnp.float32)
        ea_all = ea_all.reshape(ea_all.shape[0], n, n)
        xT = _gat_layer_fm(xT, ea_all[0:heads], mask, W1_ref[...],
                           ws1_ref[...], wd1_ref[...], b1_ref[...], heads, chp)
        xT = _gat_layer_fm(xT, ea_all[heads:2 * heads], mask, W2_ref[...],
                           ws2_ref[...], wd2_ref[...], b2_ref[...], heads, chp)
        xT = _gat_layer_fm(xT, ea_all[2 * heads:2 * heads + 1], mask,
                           W3_ref[...], ws3_ref[...], wd3_ref[...],
                           b3_ref[...], 1, chp)
        out_ref[g] = xT[:out_dim].T


def _fold(W, a):
    heads, ch = a.shape
    return jnp.einsum('dhc,hc->dh', W.reshape(W.shape[0], heads, ch), a)


def kernel(atoms, adjs, edges, W1, We1, as1, ad1, ae1, b1,
           W2, We2, as2, ad2, ae2, b2, W3, We3, as3, ad3, ae3, b3):
    Bb, n, dim = atoms.shape
    E = edges.shape[-1]
    heads, ch = as1.shape
    out_dim = W3.shape[1]
    chp = ((ch + 7) // 8) * 8
    outp = ((out_dim + 7) // 8) * 8
    pad_c = chp - ch

    W1pT = jnp.pad(W1.reshape(dim, heads, ch),
                   ((0, 0), (0, 0), (0, pad_c))).reshape(dim, heads * chp).T
    ws1p, wd1p = _fold(W1, as1), _fold(W1, ad1)
    b1p = jnp.pad(b1.reshape(heads, ch),
                  ((0, 0), (0, pad_c))).reshape(heads * chp)[:, None]
    W2p = jnp.pad(W2.reshape(heads, ch, heads, ch),
                  ((0, 0), (0, pad_c), (0, 0), (0, pad_c)))
    W2pT = W2p.reshape(heads * chp, heads * chp).T
    ws2p = jnp.pad(_fold(W2, as2).reshape(heads, ch, heads),
                   ((0, 0), (0, pad_c), (0, 0))).reshape(heads * chp, heads)
    wd2p = jnp.pad(_fold(W2, ad2).reshape(heads, ch, heads),
                   ((0, 0), (0, pad_c), (0, 0))).reshape(heads * chp, heads)
    b2p = jnp.pad(b2.reshape(heads, ch),
                  ((0, 0), (0, pad_c))).reshape(heads * chp)[:, None]
    W3p = jnp.pad(W3.reshape(heads, ch, out_dim),
                  ((0, 0), (0, pad_c), (0, 0))).reshape(heads * chp, out_dim)
    W3pT = jnp.pad(W3p, ((0, 0), (0, outp - out_dim))).T
    ws3p = jnp.pad(_fold(W3, as3).reshape(heads, ch, 1),
                   ((0, 0), (0, pad_c), (0, 0))).reshape(heads * chp, 1)
    wd3p = jnp.pad(_fold(W3, ad3).reshape(heads, ch, 1),
                   ((0, 0), (0, pad_c), (0, 0))).reshape(heads * chp, 1)
    b3p = jnp.pad(b3, (0, outp - out_dim))[:, None]
    V = jnp.concatenate([_fold(We1, ae1), _fold(We2, ae2), _fold(We3, ae3)],
                        axis=1)
    M = V.T.astype(jnp.bfloat16)

    atomsT = jnp.swapaxes(atoms, 1, 2)

    def full(x):
        return pl.BlockSpec(x.shape, lambda b: (0,) * x.ndim)

    weights = (W1pT, ws1p, wd1p, b1p, W2pT, ws2p, wd2p, b2p,
               W3pT, ws3p, wd3p, b3p, M)
    import functools
    body = functools.partial(_body, heads=heads, chp=chp, out_dim=out_dim)

    et = jnp.transpose(edges.astype(jnp.bfloat16),
                       (0, 3, 1, 2)).reshape(Bb, E, n * n)
    gpp = 2
    out = pl.pallas_call(
        body,
        grid=(Bb // gpp,),
        in_specs=[
            pl.BlockSpec((gpp, dim, n), lambda b: (b, 0, 0)),
            pl.BlockSpec((gpp, n, n), lambda b: (b, 0, 0)),
            pl.BlockSpec((gpp, E, n * n), lambda b: (b, 0, 0)),
        ] + [full(w) for w in weights],
        out_specs=pl.BlockSpec((gpp, n, out_dim), lambda b: (b, 0, 0)),
        out_shape=jax.ShapeDtypeStruct((Bb, n, out_dim), jnp.float32),
        compiler_params=pltpu.CompilerParams(
            dimension_semantics=("parallel",)),
        interpret=_INTERPRET,
    )(atomsT, adjs, et, *weights)
    return out


# speedup vs baseline: 1.2470x; 1.0109x over previous
import jax
import jax.numpy as jnp
from jax.experimental import pallas as pl
from jax.experimental.pallas import tpu as pltpu

_INTERPRET = False


def _gat_layer_fm(xT, ea_l, mask, WpT, wsp, wdp, bp, heads, chp):
    hT = jnp.dot(WpT, xT, preferred_element_type=jnp.float32)
    s = jax.lax.dot_general(xT, wsp, (((0,), (0,)), ((), ())),
                            preferred_element_type=jnp.float32)
    dT = jax.lax.dot_general(wdp, xT, (((0,), (0,)), ((), ())),
                             preferred_element_type=jnp.float32)
    outs = []
    for h in range(heads):
        logit = s[:, h][:, None] + dT[h][None, :] + ea_l[h]
        logit = jnp.maximum(logit, 0.2 * logit)
        logit = jnp.where(mask, logit, jnp.float32(-1e9))
        m = jnp.max(logit, axis=0, keepdims=True)
        e = jnp.where(mask, jnp.exp(logit - m), 0.0)
        denom = jnp.maximum(jnp.sum(e, axis=0, keepdims=True),
                            jnp.float32(1e-30))
        oh = jnp.dot(hT[h * chp:(h + 1) * chp], e,
                     preferred_element_type=jnp.float32)
        outs.append(oh / denom)
    return jnp.concatenate(outs, axis=0) + bp


def _body(atomsT_ref, adjs_ref, et_ref,
          W1_ref, ws1_ref, wd1_ref, b1_ref,
          W2_ref, ws2_ref, wd2_ref, b2_ref,
          W3_ref, ws3_ref, wd3_ref, b3_ref,
          M_ref, out_ref, *, heads, chp, out_dim):
    for g in range(out_ref.shape[0]):
        xT = atomsT_ref[g]
        n = xT.shape[1]
        mask = adjs_ref[g] > 0.5
        ea_all = jax.lax.dot_general(M_ref[...], et_ref[g],
                                     (((1,), (0,)), ((), ())),
                                     preferred_element_type=jnp.float32)
        ea_all = ea_all.reshape(ea_all.shape[0], n, n)
        xT = _gat_layer_fm(xT, ea_all[0:heads], mask, W1_ref[...],
                           ws1_ref[...], wd1_ref[...], b1_ref[...], heads, chp)
        xT = _gat_layer_fm(xT, ea_all[heads:2 * heads], mask, W2_ref[...],
                           ws2_ref[...], wd2_ref[...], b2_ref[...], heads, chp)
        xT = _gat_layer_fm(xT, ea_all[2 * heads:2 * heads + 1], mask,
                           W3_ref[...], ws3_ref[...], wd3_ref[...],
                           b3_ref[...], 1, chp)
        out_ref[g] = xT[:out_dim].T


def _fold(W, a):
    heads, ch = a.shape
    return jnp.einsum('dhc,hc->dh', W.reshape(W.shape[0], heads, ch), a)


def kernel(atoms, adjs, edges, W1, We1, as1, ad1, ae1, b1,
           W2, We2, as2, ad2, ae2, b2, W3, We3, as3, ad3, ae3, b3):
    Bb, n, dim = atoms.shape
    E = edges.shape[-1]
    heads, ch = as1.shape
    out_dim = W3.shape[1]
    chp = ((ch + 7) // 8) * 8
    outp = ((out_dim + 7) // 8) * 8
    pad_c = chp - ch

    W1pT = jnp.pad(W1.reshape(dim, heads, ch),
                   ((0, 0), (0, 0), (0, pad_c))).reshape(dim, heads * chp).T
    ws1p, wd1p = _fold(W1, as1), _fold(W1, ad1)
    b1p = jnp.pad(b1.reshape(heads, ch),
                  ((0, 0), (0, pad_c))).reshape(heads * chp)[:, None]
    W2p = jnp.pad(W2.reshape(heads, ch, heads, ch),
                  ((0, 0), (0, pad_c), (0, 0), (0, pad_c)))
    W2pT = W2p.reshape(heads * chp, heads * chp).T
    ws2p = jnp.pad(_fold(W2, as2).reshape(heads, ch, heads),
                   ((0, 0), (0, pad_c), (0, 0))).reshape(heads * chp, heads)
    wd2p = jnp.pad(_fold(W2, ad2).reshape(heads, ch, heads),
                   ((0, 0), (0, pad_c), (0, 0))).reshape(heads * chp, heads)
    b2p = jnp.pad(b2.reshape(heads, ch),
                  ((0, 0), (0, pad_c))).reshape(heads * chp)[:, None]
    W3p = jnp.pad(W3.reshape(heads, ch, out_dim),
                  ((0, 0), (0, pad_c), (0, 0))).reshape(heads * chp, out_dim)
    W3pT = jnp.pad(W3p, ((0, 0), (0, outp - out_dim))).T
    ws3p = jnp.pad(_fold(W3, as3).reshape(heads, ch, 1),
                   ((0, 0), (0, pad_c), (0, 0))).reshape(heads * chp, 1)
    wd3p = jnp.pad(_fold(W3, ad3).reshape(heads, ch, 1),
                   ((0, 0), (0, pad_c), (0, 0))).reshape(heads * chp, 1)
    b3p = jnp.pad(b3, (0, outp - out_dim))[:, None]
    V = jnp.concatenate([_fold(We1, ae1), _fold(We2, ae2), _fold(We3, ae3)],
                        axis=1)
    M = V.T.astype(jnp.bfloat16)

    atomsT = jnp.swapaxes(atoms, 1, 2)

    def full(x):
        return pl.BlockSpec(x.shape, lambda b: (0,) * x.ndim)

    weights = (W1pT, ws1p, wd1p, b1p, W2pT, ws2p, wd2p, b2p,
               W3pT, ws3p, wd3p, b3p, M)
    import functools
    body = functools.partial(_body, heads=heads, chp=chp, out_dim=out_dim)

    et = jnp.transpose(edges.astype(jnp.bfloat16),
                       (0, 3, 1, 2)).reshape(Bb, E, n * n)
    gpp = 4
    out = pl.pallas_call(
        body,
        grid=(Bb // gpp,),
        in_specs=[
            pl.BlockSpec((gpp, dim, n), lambda b: (b, 0, 0)),
            pl.BlockSpec((gpp, n, n), lambda b: (b, 0, 0)),
            pl.BlockSpec((gpp, E, n * n), lambda b: (b, 0, 0)),
        ] + [full(w) for w in weights],
        out_specs=pl.BlockSpec((gpp, n, out_dim), lambda b: (b, 0, 0)),
        out_shape=jax.ShapeDtypeStruct((Bb, n, out_dim), jnp.float32),
        compiler_params=pltpu.CompilerParams(
            dimension_semantics=("parallel",)),
        interpret=_INTERPRET,
    )(atomsT, adjs, et, *weights)
    return out


# speedup vs baseline: 1.2500x; 1.0024x over previous
import jax
import jax.numpy as jnp
from jax.experimental import pallas as pl
from jax.experimental.pallas import tpu as pltpu

_INTERPRET = False


def _gat_layer_fm(xT, ea_l, mask, WpT, wsp, wdp, bp, heads, chp):
    hT = jnp.dot(WpT, xT, preferred_element_type=jnp.float32)
    s = jax.lax.dot_general(xT, wsp, (((0,), (0,)), ((), ())),
                            preferred_element_type=jnp.float32)
    dT = jax.lax.dot_general(wdp, xT, (((0,), (0,)), ((), ())),
                             preferred_element_type=jnp.float32)
    outs = []
    for h in range(heads):
        logit = s[:, h][:, None] + dT[h][None, :] + ea_l[h]
        logit = jnp.maximum(logit, 0.2 * logit)
        logit = jnp.where(mask, logit, jnp.float32(-1e9))
        m = jnp.max(logit, axis=0, keepdims=True)
        e = jnp.where(mask, jnp.exp(logit - m), 0.0)
        denom = jnp.maximum(jnp.sum(e, axis=0, keepdims=True),
                            jnp.float32(1e-30))
        oh = jnp.dot(hT[h * chp:(h + 1) * chp], e,
                     preferred_element_type=jnp.float32)
        outs.append(oh / denom)
    return jnp.concatenate(outs, axis=0) + bp


def _body(atomsT_ref, adjs_ref, et_ref,
          W1_ref, ws1_ref, wd1_ref, b1_ref,
          W2_ref, ws2_ref, wd2_ref, b2_ref,
          W3_ref, ws3_ref, wd3_ref, b3_ref,
          M_ref, out_ref, *, heads, chp, out_dim):
    for g in range(out_ref.shape[0]):
        xT = atomsT_ref[g]
        n = xT.shape[1]
        mask = adjs_ref[g] > 0.5
        ea_all = jax.lax.dot_general(M_ref[...], et_ref[g],
                                     (((1,), (0,)), ((), ())),
                                     preferred_element_type=jnp.float32)
        ea_all = ea_all.reshape(ea_all.shape[0], n, n)
        xT = _gat_layer_fm(xT, ea_all[0:heads], mask, W1_ref[...],
                           ws1_ref[...], wd1_ref[...], b1_ref[...], heads, chp)
        xT = _gat_layer_fm(xT, ea_all[heads:2 * heads], mask, W2_ref[...],
                           ws2_ref[...], wd2_ref[...], b2_ref[...], heads, chp)
        xT = _gat_layer_fm(xT, ea_all[2 * heads:2 * heads + 1], mask,
                           W3_ref[...], ws3_ref[...], wd3_ref[...],
                           b3_ref[...], 1, chp)
        out_ref[g] = xT[:out_dim].T


def _fold(W, a):
    heads, ch = a.shape
    return jnp.einsum('dhc,hc->dh', W.reshape(W.shape[0], heads, ch), a)


def kernel(atoms, adjs, edges, W1, We1, as1, ad1, ae1, b1,
           W2, We2, as2, ad2, ae2, b2, W3, We3, as3, ad3, ae3, b3):
    Bb, n, dim = atoms.shape
    E = edges.shape[-1]
    heads, ch = as1.shape
    out_dim = W3.shape[1]
    chp = ((ch + 7) // 8) * 8
    outp = ((out_dim + 7) // 8) * 8
    pad_c = chp - ch

    W1pT = jnp.pad(W1.reshape(dim, heads, ch),
                   ((0, 0), (0, 0), (0, pad_c))).reshape(dim, heads * chp).T
    ws1p, wd1p = _fold(W1, as1), _fold(W1, ad1)
    b1p = jnp.pad(b1.reshape(heads, ch),
                  ((0, 0), (0, pad_c))).reshape(heads * chp)[:, None]
    W2p = jnp.pad(W2.reshape(heads, ch, heads, ch),
                  ((0, 0), (0, pad_c), (0, 0), (0, pad_c)))
    W2pT = W2p.reshape(heads * chp, heads * chp).T
    ws2p = jnp.pad(_fold(W2, as2).reshape(heads, ch, heads),
                   ((0, 0), (0, pad_c), (0, 0))).reshape(heads * chp, heads)
    wd2p = jnp.pad(_fold(W2, ad2).reshape(heads, ch, heads),
                   ((0, 0), (0, pad_c), (0, 0))).reshape(heads * chp, heads)
    b2p = jnp.pad(b2.reshape(heads, ch),
                  ((0, 0), (0, pad_c))).reshape(heads * chp)[:, None]
    W3p = jnp.pad(W3.reshape(heads, ch, out_dim),
                  ((0, 0), (0, pad_c), (0, 0))).reshape(heads * chp, out_dim)
    W3pT = jnp.pad(W3p, ((0, 0), (0, outp - out_dim))).T
    ws3p = jnp.pad(_fold(W3, as3).reshape(heads, ch, 1),
                   ((0, 0), (0, pad_c), (0, 0))).reshape(heads * chp, 1)
    wd3p = jnp.pad(_fold(W3, ad3).reshape(heads, ch, 1),
                   ((0, 0), (0, pad_c), (0, 0))).reshape(heads * chp, 1)
    b3p = jnp.pad(b3, (0, outp - out_dim))[:, None]
    V = jnp.concatenate([_fold(We1, ae1), _fold(We2, ae2), _fold(We3, ae3)],
                        axis=1)
    M = V.T.astype(jnp.bfloat16)

    atomsT = jnp.swapaxes(atoms, 1, 2)

    def full(x):
        return pl.BlockSpec(x.shape, lambda b: (0,) * x.ndim)

    weights = (W1pT, ws1p, wd1p, b1p, W2pT, ws2p, wd2p, b2p,
               W3pT, ws3p, wd3p, b3p, M)
    import functools
    body = functools.partial(_body, heads=heads, chp=chp, out_dim=out_dim)

    et = jnp.transpose(edges.astype(jnp.bfloat16),
                       (0, 3, 1, 2)).reshape(Bb, E, n * n)
    gpp = 8
    out = pl.pallas_call(
        body,
        grid=(Bb // gpp,),
        in_specs=[
            pl.BlockSpec((gpp, dim, n), lambda b: (b, 0, 0)),
            pl.BlockSpec((gpp, n, n), lambda b: (b, 0, 0)),
            pl.BlockSpec((gpp, E, n * n), lambda b: (b, 0, 0)),
        ] + [full(w) for w in weights],
        out_specs=pl.BlockSpec((gpp, n, out_dim), lambda b: (b, 0, 0)),
        out_shape=jax.ShapeDtypeStruct((Bb, n, out_dim), jnp.float32),
        compiler_params=pltpu.CompilerParams(
            dimension_semantics=("parallel",)),
        interpret=_INTERPRET,
    )(atomsT, adjs, et, *weights)
    return out
